# baseline (speedup 1.0000x reference)
import jax
import jax.numpy as jnp
from jax.experimental import pallas as pl
from jax.experimental.pallas import tpu as pltpu


def _rvq_kernel(z_ref, wpre_ref, bpre_ref, cbs_ref, cb2_ref, cbcat_ref,
                wpost_ref, bpost_ref, out_ref, idx_ref, loss_ref, c2_ref):
    b = pl.program_id(0)
    t = pl.program_id(1)
    ncb, kk, dd = cbs_ref.shape
    tt = z_ref.shape[2]
    th = 256
    nh = tt // th

    @pl.when((b == 0) & (t == 0))
    def _precompute():
        for i in range(ncb):
            cb = cbs_ref[i]
            c2_ref[i:i + 1, :] = jnp.sum(cb * cb, axis=1)[None, :]

    z = z_ref[0]
    wpre = wpre_ref[...]
    xp = jax.lax.dot_general(z, wpre, (((0,), (1,)), ((), ())),
                             preferred_element_type=jnp.float32)
    res0 = xp + bpre_ref[...]

    halves = [res0[j * th:(j + 1) * th] for j in range(nh)]
    loss = jnp.float32(0.0)
    iota_f = jax.lax.broadcasted_iota(jnp.int32, (th, kk), 1).astype(jnp.float32)
    for i in range(ncb):
        cb2 = cb2_ref[i]
        cbcat = cbcat_ref[i]
        c2 = c2_ref[i:i + 1, :]
        for h in range(nh):
            res = halves[h]
            s2 = jax.lax.dot_general(res, cb2, (((1,), (1,)), ((), ())),
                                     preferred_element_type=jnp.float32)
            a = jnp.sum(res * res, axis=1, keepdims=True)
            d2 = a - s2 + c2
            minv = jnp.min(d2, axis=1, keepdims=True)
            sel = jnp.where(d2 == minv, iota_f, jnp.float32(kk))
            idxf = jnp.min(sel, axis=1, keepdims=True)
            oh = (sel == idxf).astype(jnp.bfloat16)
            dn = (((1,), (0,)), ((), ()))
            qcat = jax.lax.dot_general(oh, cbcat, dn,
                                       preferred_element_type=jnp.float32)
            q = (qcat[:, :dd] + qcat[:, dd:2 * dd]) + qcat[:, 2 * dd:]
            rn = res - q
            loss += jnp.sum(rn * rn)
            halves[h] = rn
            idx_ref[0, pl.ds(h * th, th), i:i + 1] = idxf.astype(jnp.int32)

    qsum = res0 - jnp.concatenate(halves, axis=0)
    o = jax.lax.dot_general(wpost_ref[...], qsum, (((1,), (1,)), ((), ())),
                            preferred_element_type=jnp.float32)
    out_ref[0] = o + bpost_ref[...]

    @pl.when((b == 0) & (t == 0))
    def _init():
        loss_ref[...] = jnp.zeros_like(loss_ref)
    loss_ref[...] = loss_ref[...] + loss


def kernel(z, W_pre, b_pre, codebooks, W_post, b_post):
    Bz, Hz, Tz = z.shape
    Dd = W_pre.shape[0]
    ncb, kk, _ = codebooks.shape
    tt = min(2048, Tz)
    grid = (Bz, Tz // tt)

    msk = jnp.uint32(0xFFFF0000)
    bc = jax.lax.bitcast_convert_type
    cb_hi = bc(bc(codebooks, jnp.uint32) & msk, jnp.float32)
    r1 = codebooks - cb_hi
    cb_mid = bc(bc(r1, jnp.uint32) & msk, jnp.float32)
    cb_lo = r1 - cb_mid
    cbcat = jnp.concatenate([cb_hi.astype(jnp.bfloat16),
                             cb_mid.astype(jnp.bfloat16),
                             cb_lo.astype(jnp.bfloat16)], axis=-1)
    cb2 = codebooks * 2.0

    full = lambda b, t: (0, 0)
    full3 = lambda b, t: (0, 0, 0)
    out, idx, loss_sum = pl.pallas_call(
        _rvq_kernel,
        grid=grid,
        in_specs=[
            pl.BlockSpec((1, Hz, tt), lambda b, t: (b, 0, t)),
            pl.BlockSpec((Dd, Hz), full),
            pl.BlockSpec((1, Dd), full),
            pl.BlockSpec((ncb, kk, Dd), full3),
            pl.BlockSpec((ncb, kk, Dd), full3),
            pl.BlockSpec((ncb, kk, 3 * Dd), full3),
            pl.BlockSpec((Hz, Dd), full),
            pl.BlockSpec((Hz, 1), full),
        ],
        out_specs=(
            pl.BlockSpec((1, Hz, tt), lambda b, t: (b, 0, t)),
            pl.BlockSpec((1, tt, ncb), lambda b, t: (b, t, 0)),
            pl.BlockSpec((1, 1), lambda b, t: (0, 0)),
        ),
        out_shape=(
            jax.ShapeDtypeStruct((Bz, Hz, Tz), jnp.float32),
            jax.ShapeDtypeStruct((Bz, Tz, ncb), jnp.int32),
            jax.ShapeDtypeStruct((1, 1), jnp.float32),
        ),
        scratch_shapes=[pltpu.VMEM((ncb, kk), jnp.float32)],
    )(z, W_pre, b_pre.reshape(1, Dd), codebooks, cb2, cbcat,
      W_post, b_post.reshape(Hz, 1))

    total_loss = loss_sum[0, 0] / jnp.float32(Bz * Dd * Tz)
    indices = tuple(idx[:, :, i] for i in range(ncb))
    return (out, indices, total_loss)


# speedup vs baseline: 1.2882x; 1.2882x over previous
import jax
import jax.numpy as jnp
from jax.experimental import pallas as pl
from jax.experimental.pallas import tpu as pltpu


def _rvq_kernel(z_ref, wpre_ref, bpre_ref, cbs_ref, cb2_ref, cbcat_ref,
                wpost_ref, bpost_ref, out_ref, idx_ref, loss_ref, c2_ref):
    b = pl.program_id(0)
    t = pl.program_id(1)
    ncb, kk, dd = cbs_ref.shape
    tt = z_ref.shape[2]
    th = 1024
    nh = tt // th

    @pl.when((b == 0) & (t == 0))
    def _precompute():
        for i in range(ncb):
            cb = cbs_ref[i]
            c2_ref[i:i + 1, :] = jnp.sum(cb * cb, axis=1)[None, :]

    z = z_ref[0]
    wpre = wpre_ref[...]
    xp = jax.lax.dot_general(z, wpre, (((0,), (1,)), ((), ())),
                             preferred_element_type=jnp.float32)
    res0 = xp + bpre_ref[...]

    halves = [res0[j * th:(j + 1) * th] for j in range(nh)]
    loss = jnp.float32(0.0)
    iota_f = jax.lax.broadcasted_iota(jnp.int32, (th, kk), 1).astype(jnp.float32)
    for i in range(ncb):
        cb2 = cb2_ref[i]
        cbcat = cbcat_ref[i]
        c2 = c2_ref[i:i + 1, :]
        for h in range(nh):
            res = halves[h]
            s2 = jax.lax.dot_general(res, cb2, (((1,), (1,)), ((), ())),
                                     preferred_element_type=jnp.float32)
            a = jnp.sum(res * res, axis=1, keepdims=True)
            d2 = a - s2 + c2
            minv = jnp.min(d2, axis=1, keepdims=True)
            sel = jnp.where(d2 == minv, iota_f, jnp.float32(kk))
            idxf = jnp.min(sel, axis=1, keepdims=True)
            oh = (sel == idxf).astype(jnp.bfloat16)
            dn = (((1,), (0,)), ((), ()))
            qcat = jax.lax.dot_general(oh, cbcat, dn,
                                       preferred_element_type=jnp.float32)
            q = (qcat[:, :dd] + qcat[:, dd:2 * dd]) + qcat[:, 2 * dd:]
            rn = res - q
            loss += jnp.sum(rn * rn)
            halves[h] = rn
            idx_ref[0, pl.ds(h * th, th), i:i + 1] = idxf.astype(jnp.int32)

    qsum = res0 - jnp.concatenate(halves, axis=0)
    o = jax.lax.dot_general(wpost_ref[...], qsum, (((1,), (1,)), ((), ())),
                            preferred_element_type=jnp.float32)
    out_ref[0] = o + bpost_ref[...]

    @pl.when((b == 0) & (t == 0))
    def _init():
        loss_ref[...] = jnp.zeros_like(loss_ref)
    loss_ref[...] = loss_ref[...] + loss


def kernel(z, W_pre, b_pre, codebooks, W_post, b_post):
    Bz, Hz, Tz = z.shape
    Dd = W_pre.shape[0]
    ncb, kk, _ = codebooks.shape
    tt = min(2048, Tz)
    grid = (Bz, Tz // tt)

    msk = jnp.uint32(0xFFFF0000)
    bc = jax.lax.bitcast_convert_type
    cb_hi = bc(bc(codebooks, jnp.uint32) & msk, jnp.float32)
    r1 = codebooks - cb_hi
    cb_mid = bc(bc(r1, jnp.uint32) & msk, jnp.float32)
    cb_lo = r1 - cb_mid
    cbcat = jnp.concatenate([cb_hi.astype(jnp.bfloat16),
                             cb_mid.astype(jnp.bfloat16),
                             cb_lo.astype(jnp.bfloat16)], axis=-1)
    cb2 = codebooks * 2.0

    full = lambda b, t: (0, 0)
    full3 = lambda b, t: (0, 0, 0)
    out, idx, loss_sum = pl.pallas_call(
        _rvq_kernel,
        grid=grid,
        in_specs=[
            pl.BlockSpec((1, Hz, tt), lambda b, t: (b, 0, t)),
            pl.BlockSpec((Dd, Hz), full),
            pl.BlockSpec((1, Dd), full),
            pl.BlockSpec((ncb, kk, Dd), full3),
            pl.BlockSpec((ncb, kk, Dd), full3),
            pl.BlockSpec((ncb, kk, 3 * Dd), full3),
            pl.BlockSpec((Hz, Dd), full),
            pl.BlockSpec((Hz, 1), full),
        ],
        out_specs=(
            pl.BlockSpec((1, Hz, tt), lambda b, t: (b, 0, t)),
            pl.BlockSpec((1, tt, ncb), lambda b, t: (b, t, 0)),
            pl.BlockSpec((1, 1), lambda b, t: (0, 0)),
        ),
        out_shape=(
            jax.ShapeDtypeStruct((Bz, Hz, Tz), jnp.float32),
            jax.ShapeDtypeStruct((Bz, Tz, ncb), jnp.int32),
            jax.ShapeDtypeStruct((1, 1), jnp.float32),
        ),
        scratch_shapes=[pltpu.VMEM((ncb, kk), jnp.float32)],
    )(z, W_pre, b_pre.reshape(1, Dd), codebooks, cb2, cbcat,
      W_post, b_post.reshape(Hz, 1))

    total_loss = loss_sum[0, 0] / jnp.float32(Bz * Dd * Tz)
    indices = tuple(idx[:, :, i] for i in range(ncb))
    return (out, indices, total_loss)
